# baseline (speedup 1.0000x reference)
import functools

import numpy as np
import jax
import jax.numpy as jnp
from jax import lax
from jax.experimental import pallas as pl
from jax.experimental.pallas import tpu as pltpu
from jax.experimental.pallas import tpu_sc as plsc

MAXL = 8192
BATCH = 2
D = 768
NC, NS, LANES = 2, 16, 16
NW = NC * NS
L_PER_W = MAXL // NW
PC = 16
NCHUNK = L_PER_W // PC
NGRP = D // LANES
SLOTS = 3


def _make_pos_factors():
    d = np.arange(D)
    w = 10000.0 ** (-2.0 * (d // 2) / D)
    l0 = (np.arange(NW * NCHUNK) * PC)[:, None]
    even = (d % 2 == 0)
    u = np.where(even, np.sin(l0 * w), np.cos(l0 * w))
    v = np.where(even, np.cos(l0 * w), -np.sin(l0 * w))
    uv = np.stack([u, v], axis=1).reshape(NW, NCHUNK, 2, D).astype(np.float32)

    t = np.arange(PC)[:, None]
    cs = np.stack([np.cos(t * w), np.sin(t * w)], axis=0
                  ).astype(np.float32)
    return uv, cs


_UV, _CSB = _make_pos_factors()


@functools.partial(
    pl.kernel,
    out_type=jax.ShapeDtypeStruct((MAXL, BATCH, D), jnp.float32),
    mesh=plsc.VectorSubcoreMesh(core_axis_name="c", subcore_axis_name="s"),
    scratch_types=(
        [pltpu.VMEM((NCHUNK, BATCH, PC), jnp.int32),
         pltpu.VMEM((2, PC, D), jnp.float32)]
        + [pltpu.VMEM((BATCH, PC, D), jnp.float32) for _ in range(SLOTS)]
        + [pltpu.VMEM((2, D), jnp.float32) for _ in range(SLOTS)]
        + [pltpu.SemaphoreType.DMA for _ in range(2 * SLOTS)]
    ),
)
def _emb_kernel(x_hbm, uv_hbm, cs_hbm, table_hbm, out_hbm, idx_v, cs_v,
                rows0, rows1, rows2, uv0, uv1, uv2,
                gsem0, gsem1, gsem2, osem0, osem1, osem2):
    rows = (rows0, rows1, rows2)
    uvb = (uv0, uv1, uv2)
    gsem = (gsem0, gsem1, gsem2)
    osem = (osem0, osem1, osem2)

    wid = lax.axis_index("s") * NC + lax.axis_index("c")
    lbase = wid * L_PER_W

    def start(j):
        s = j % SLOTS
        g0 = pltpu.async_copy(table_hbm.at[idx_v.at[j, 0]], rows[s].at[0],
                              gsem[s])
        g1 = pltpu.async_copy(table_hbm.at[idx_v.at[j, 1]], rows[s].at[1],
                              gsem[s])
        p = pltpu.async_copy(uv_hbm.at[wid, j], uvb[s], gsem[s])
        return (g0, g1, p)

    pltpu.sync_copy(x_hbm.at[wid], idx_v)

    descs = [None] * NCHUNK
    odescs = [None] * NCHUNK
    descs[0] = start(0)
    descs[1] = start(1)

    pltpu.sync_copy(cs_hbm, cs_v)

    for j in range(NCHUNK):
        s = j % SLOTS
        if j + 1 >= 2 and j + 1 < NCHUNK:
            if j - 2 >= 0:
                for od in odescs[j - 2]:
                    od.wait()
            descs[j + 1] = start(j + 1)

        for dsc in descs[j]:
            dsc.wait()

        rs = rows[s]
        uvs = uvb[s]
        l0 = lbase + j * PC
        HALF = PC // 2
        ods = []

        for h in range(2):
            t0 = h * HALF

            def grp_body(grp, carry, t0=t0):
                sl = pl.ds(grp * LANES, LANES)
                u = uvs[0, sl]
                v = uvs[1, sl]

                @plsc.parallel_loop(t0, t0 + HALF, unroll=4)
                def _(t):
                    pv = u * cs_v[0, t, sl] + v * cs_v[1, t, sl]
                    a = rs[0, t, sl]
                    b = rs[1, t, sl]
                    rs[0, t, sl] = a + a + pv
                    rs[1, t, sl] = b + b + pv

                return carry

            lax.fori_loop(0, NGRP, grp_body, 0)
            ods.append(pltpu.async_copy(
                rs.at[0].at[pl.ds(t0, HALF)],
                out_hbm.at[pl.ds(l0 + t0, HALF), 0], osem[s]))
            ods.append(pltpu.async_copy(
                rs.at[1].at[pl.ds(t0, HALF)],
                out_hbm.at[pl.ds(l0 + t0, HALF), 1], osem[s]))

        odescs[j] = tuple(ods)

    for j in range(NCHUNK - SLOTS, NCHUNK):
        for od in odescs[j]:
            od.wait()


def kernel(x, table):
    xi = (x.astype(jnp.int32)
          .reshape(NW, NCHUNK, PC, BATCH)
          .transpose(0, 1, 3, 2))
    return _emb_kernel(xi, jnp.asarray(_UV), jnp.asarray(_CSB), table)


# speedup vs baseline: 1.1641x; 1.1641x over previous
import functools

import numpy as np
import jax
import jax.numpy as jnp
from jax import lax
from jax.experimental import pallas as pl
from jax.experimental.pallas import tpu as pltpu
from jax.experimental.pallas import tpu_sc as plsc

MAXL = 8192
BATCH = 2
D = 768
NC, NS, LANES = 2, 16, 16
NW = NC * NS
L_PER_W = MAXL // NW
PC = 16
NCHUNK = L_PER_W // PC
NGRP = D // LANES
SLOTS = 3


def _make_pos_factors():
    d = np.arange(D)
    w = 10000.0 ** (-2.0 * (d // 2) / D)
    l0 = (np.arange(NW * NCHUNK) * PC)[:, None]
    even = (d % 2 == 0)
    u = np.where(even, np.sin(l0 * w), np.cos(l0 * w))
    v = np.where(even, np.cos(l0 * w), -np.sin(l0 * w))
    uv = np.stack([u, v], axis=1).reshape(NW, NCHUNK, 2, D).astype(np.float32)

    t = np.arange(PC)[:, None]
    cs = np.stack([np.cos(t * w), np.sin(t * w)], axis=0
                  ).astype(np.float32)
    return uv, cs


_UV, _CS = _make_pos_factors()


@functools.partial(
    pl.kernel,
    out_type=jax.ShapeDtypeStruct((MAXL, BATCH, D), jnp.float32),
    mesh=plsc.VectorSubcoreMesh(core_axis_name="c", subcore_axis_name="s"),
    scratch_types=(
        [pltpu.VMEM((NCHUNK, BATCH, PC), jnp.int32),
         pltpu.VMEM((2, PC, D), jnp.float32)]
        + [pltpu.VMEM((BATCH, PC, D), jnp.float32) for _ in range(SLOTS)]
        + [pltpu.VMEM((2, D), jnp.float32) for _ in range(SLOTS)]
        + [pltpu.SemaphoreType.DMA for _ in range(2 * SLOTS)]
    ),
)
def _emb_kernel(x_hbm, uv_hbm, cs_hbm, table_hbm, out_hbm, idx_v, cs_v,
                rows0, rows1, rows2, uv0, uv1, uv2,
                gsem0, gsem1, gsem2, osem0, osem1, osem2):
    rows = (rows0, rows1, rows2)
    uvb = (uv0, uv1, uv2)
    gsem = (gsem0, gsem1, gsem2)
    osem = (osem0, osem1, osem2)

    wid = lax.axis_index("s") * NC + lax.axis_index("c")
    lbase = wid * L_PER_W

    def start(j):
        s = j % SLOTS
        g0 = pltpu.async_copy(table_hbm.at[idx_v.at[j, 0]], rows[s].at[0],
                              gsem[s])
        g1 = pltpu.async_copy(table_hbm.at[idx_v.at[j, 1]], rows[s].at[1],
                              gsem[s])
        p = pltpu.async_copy(uv_hbm.at[wid, j], uvb[s], gsem[s])
        return (g0, g1, p)

    pltpu.sync_copy(x_hbm.at[wid], idx_v)

    descs = [None] * NCHUNK
    odescs = [None] * NCHUNK
    descs[0] = start(0)
    descs[1] = start(1)

    pltpu.sync_copy(cs_hbm, cs_v)

    for j in range(NCHUNK):
        s = j % SLOTS
        if j + 1 >= 2 and j + 1 < NCHUNK:
            if j - 2 >= 0:
                for od in odescs[j - 2]:
                    od.wait()
            descs[j + 1] = start(j + 1)

        for dsc in descs[j]:
            dsc.wait()

        rs = rows[s]
        uvs = uvb[s]

        def grp_body(grp, carry):
            sl = pl.ds(grp * LANES, LANES)
            u = uvs[0, sl]
            v = uvs[1, sl]

            @plsc.parallel_loop(0, PC, unroll=4)
            def _(t):
                pv = u * cs_v[0, t, sl] + v * cs_v[1, t, sl]
                a = rs[0, t, sl]
                b = rs[1, t, sl]
                rs[0, t, sl] = a + a + pv
                rs[1, t, sl] = b + b + pv

            return carry

        lax.fori_loop(0, NGRP, grp_body, 0)

        l0 = lbase + j * PC
        odescs[j] = (
            pltpu.async_copy(rs.at[0], out_hbm.at[pl.ds(l0, PC), 0], osem[s]),
            pltpu.async_copy(rs.at[1], out_hbm.at[pl.ds(l0, PC), 1], osem[s]),
        )

    for j in range(NCHUNK - SLOTS, NCHUNK):
        for od in odescs[j]:
            od.wait()


def kernel(x, table):
    xi = (x.astype(jnp.int32)
          .reshape(NW, NCHUNK, PC, BATCH)
          .transpose(0, 1, 3, 2))
    return _emb_kernel(xi, jnp.asarray(_UV), jnp.asarray(_CS), table)


# speedup vs baseline: 1.1652x; 1.0010x over previous
import functools

import numpy as np
import jax
import jax.numpy as jnp
from jax import lax
from jax.experimental import pallas as pl
from jax.experimental.pallas import tpu as pltpu
from jax.experimental.pallas import tpu_sc as plsc

MAXL = 8192
BATCH = 2
D = 768
NC, NS, LANES = 2, 16, 16
NW = NC * NS
L_PER_W = MAXL // NW
PC = 16
NCHUNK = L_PER_W // PC
NGRP = D // LANES
SLOTS = 3


def _make_pos_factors():
    d = np.arange(D)
    w = 10000.0 ** (-2.0 * (d // 2) / D)
    l0 = (np.arange(NW * NCHUNK) * PC)[:, None]
    even = (d % 2 == 0)
    u = np.where(even, np.sin(l0 * w), np.cos(l0 * w))
    v = np.where(even, np.cos(l0 * w), -np.sin(l0 * w))
    uv = np.stack([u, v], axis=1).reshape(NW, NCHUNK, 2, D).astype(np.float32)

    t = np.arange(PC)[:, None]
    cs = np.stack([np.cos(t * w), np.sin(t * w)], axis=0
                  ).astype(np.float32)
    return uv, cs


_UV, _CS = _make_pos_factors()


@functools.partial(
    pl.kernel,
    out_type=jax.ShapeDtypeStruct((MAXL, BATCH, D), jnp.float32),
    mesh=plsc.VectorSubcoreMesh(core_axis_name="c", subcore_axis_name="s"),
    scratch_types=(
        [pltpu.VMEM((NCHUNK, BATCH, PC), jnp.int32),
         pltpu.VMEM((2, PC, D), jnp.float32)]
        + [pltpu.VMEM((BATCH, PC, D), jnp.float32) for _ in range(SLOTS)]
        + [pltpu.VMEM((2, D), jnp.float32) for _ in range(SLOTS)]
        + [pltpu.SemaphoreType.DMA for _ in range(2 * SLOTS)]
    ),
)
def _emb_kernel(x_hbm, uv_hbm, cs_hbm, table_hbm, out_hbm, idx_v, cs_v,
                rows0, rows1, rows2, uv0, uv1, uv2,
                gsem0, gsem1, gsem2, osem0, osem1, osem2):
    rows = (rows0, rows1, rows2)
    uvb = (uv0, uv1, uv2)
    gsem = (gsem0, gsem1, gsem2)
    osem = (osem0, osem1, osem2)

    wid = lax.axis_index("s") * NC + lax.axis_index("c")
    lbase = wid * L_PER_W

    def start(j):
        s = j % SLOTS
        g0 = pltpu.async_copy(table_hbm.at[idx_v.at[j, 0]], rows[s].at[0],
                              gsem[s])
        g1 = pltpu.async_copy(table_hbm.at[idx_v.at[j, 1]], rows[s].at[1],
                              gsem[s])
        p = pltpu.async_copy(uv_hbm.at[wid, j], uvb[s], gsem[s])
        return (g0, g1, p)

    pltpu.sync_copy(x_hbm.at[wid], idx_v)

    descs = [None] * NCHUNK
    odescs = [None] * NCHUNK
    descs[0] = start(0)
    descs[1] = start(1)

    pltpu.sync_copy(cs_hbm, cs_v)

    for j in range(NCHUNK):
        s = j % SLOTS
        if j + 1 >= 2 and j + 1 < NCHUNK:
            if j - 2 >= 0:
                for od in odescs[j - 2]:
                    od.wait()
            descs[j + 1] = start(j + 1)

        for dsc in descs[j]:
            dsc.wait()

        rs = rows[s]
        uvs = uvb[s]

        @plsc.parallel_loop(0, NGRP)
        def _grp(grp):
            sl = pl.ds(grp * LANES, LANES)
            u = uvs[0, sl]
            v = uvs[1, sl]

            @plsc.parallel_loop(0, PC, unroll=4)
            def _(t):
                pv = u * cs_v[0, t, sl] + v * cs_v[1, t, sl]
                a = rs[0, t, sl]
                b = rs[1, t, sl]
                rs[0, t, sl] = a + a + pv
                rs[1, t, sl] = b + b + pv

        l0 = lbase + j * PC
        odescs[j] = (
            pltpu.async_copy(rs.at[0], out_hbm.at[pl.ds(l0, PC), 0], osem[s]),
            pltpu.async_copy(rs.at[1], out_hbm.at[pl.ds(l0, PC), 1], osem[s]),
        )

    for j in range(NCHUNK - SLOTS, NCHUNK):
        for od in odescs[j]:
            od.wait()


def kernel(x, table):
    xi = (x.astype(jnp.int32)
          .reshape(NW, NCHUNK, PC, BATCH)
          .transpose(0, 1, 3, 2))
    return _emb_kernel(xi, jnp.asarray(_UV), jnp.asarray(_CS), table)
